# baseline (speedup 1.0000x reference)
import jax
import jax.numpy as jnp
from jax import lax
from jax.experimental import pallas as pl
from jax.experimental.pallas import tpu as pltpu
from jax.experimental.pallas import tpu_sc as plsc
from jax.experimental.layout import Format, Layout

B = 4096
L = 200
NE = 2
J = L + 2 * NE
H0 = 104
H1 = L - H0
DW = 64
DP = 16
DOUT = DW + 2 * DP
RB = 16

_info = plsc.get_sparse_core_info()
NC = _info.num_cores
NS = _info.num_subcores
NW = NC * NS
ROWS_PER_W = B // NW


def _body(seq_hbm, e1_hbm, e2_hbm, we_hbm, wpe_hbm, out_hbm,
          seq_rows, e1_rows, e2_rows, we0,
          wb0, pb10, pb20, wb1, pb11, pb21,
          sg0, sg1, sw0, sw1):
    sets = ((wb0, pb10, pb20, sg0, sw0),
            (wb1, pb11, pb21, sg1, sw1))
    wid = lax.axis_index("s") * NC + lax.axis_index("c")
    b0 = wid * ROWS_PER_W

    pltpu.sync_copy(we_hbm.at[pl.ds(0, 1)], we0)
    for k in range(DW // 16):
        v = we0[0, pl.ds(k * 16, 16)]
        for wbx in (wb0, wb1):
            for rr in (0, 1, J - 2, J - 1):
                wbx[rr, pl.ds(k * 16, 16)] = v

    def gathers(s, r):
        wbx, p1x, p2x, sg = sets[s][0], sets[s][1], sets[s][2], sets[s][3]
        g = lax.rem(r, RB)
        cps = []
        for (off, n) in ((0, H0), (H0, H1)):
            cps.append(pltpu.make_async_copy(
                we_hbm.at[seq_rows.at[g, pl.ds(off, n)]],
                wbx.at[pl.ds(NE + off, n)], sg))
            cps.append(pltpu.make_async_copy(
                wpe_hbm.at[e1_rows.at[g, pl.ds(off, n)]],
                p1x.at[pl.ds(NE + off, n)], sg))
            cps.append(pltpu.make_async_copy(
                wpe_hbm.at[e2_rows.at[g, pl.ds(off, n)]],
                p2x.at[pl.ds(NE + off, n)], sg))
        return cps

    def fixup(s):
        for px in (sets[s][1], sets[s][2]):
            v = px[NE]
            px[0] = v
            px[1] = v
            v = px[J - 3]
            px[J - 2] = v
            px[J - 1] = v

    def writes(s, b):
        wbx, p1x, p2x, sw = sets[s][0], sets[s][1], sets[s][2], sets[s][4]
        return [
            pltpu.make_async_copy(wbx, out_hbm.at[b, :, pl.ds(0, DW)], sw),
            pltpu.make_async_copy(p1x, out_hbm.at[b, :, pl.ds(DW, DP)], sw),
            pltpu.make_async_copy(p2x, out_hbm.at[b, :, pl.ds(DW + DP, DP)],
                                  sw),
        ]

    def stage_rows(r):
        @pl.when(lax.rem(r, RB) == 0)
        def _():
            base = pl.multiple_of(b0 + r, RB)
            pltpu.sync_copy(seq_hbm.at[pl.ds(base, RB)], seq_rows)
            pltpu.sync_copy(e1_hbm.at[pl.ds(base, RB)], e1_rows)
            pltpu.sync_copy(e2_hbm.at[pl.ds(base, RB)], e2_rows)

    def half(r, s, so, do_waitw):
        for cp in gathers(s, r):
            cp.wait()
        fixup(s)
        for cp in writes(s, b0 + r):
            cp.start()
        stage_rows(r + 1)
        if do_waitw:
            for cp in writes(so, b0 + r - 1):
                cp.wait()
        for cp in gathers(so, r + 1):
            cp.start()

    stage_rows(0)
    for cp in gathers(0, 0):
        cp.start()
    half(0, 0, 1, do_waitw=False)

    def pair(i, carry):
        r = 2 * i + 1
        half(r, 1, 0, do_waitw=True)
        half(r + 1, 0, 1, do_waitw=True)
        return carry

    lax.fori_loop(0, (ROWS_PER_W - 2) // 2, pair, 0)

    rlast = ROWS_PER_W - 1
    for cp in gathers(1, rlast):
        cp.wait()
    fixup(1)
    for cp in writes(1, b0 + rlast):
        cp.start()
    for cp in writes(0, b0 + rlast - 1):
        cp.wait()
    for cp in writes(1, b0 + rlast):
        cp.wait()


def _kernel_impl(seq_inputs, e1_pos_inputs, e2_pos_inputs, we, wpe):
    run = pl.kernel(
        _body,
        mesh=plsc.VectorSubcoreMesh(core_axis_name="c", subcore_axis_name="s"),
        compiler_params=pltpu.CompilerParams(use_tc_tiling_on_sc=False,
                                             needs_layout_passes=False),
        out_type=jax.ShapeDtypeStruct((B, J, DOUT), jnp.float32),
        scratch_types=[
            pltpu.VMEM((RB, L), jnp.int32),
            pltpu.VMEM((RB, L), jnp.int32),
            pltpu.VMEM((RB, L), jnp.int32),
            pltpu.VMEM((1, DW), jnp.float32),
        ] + 2 * [
            pltpu.VMEM((J, DW), jnp.float32),
            pltpu.VMEM((J, DP), jnp.float32),
            pltpu.VMEM((J, DP), jnp.float32),
        ] + 4 * [pltpu.SemaphoreType.DMA],
    )
    return run(seq_inputs.astype(jnp.int32),
               e1_pos_inputs.astype(jnp.int32),
               e2_pos_inputs.astype(jnp.int32),
               we, wpe)


_jitted_cache = []


def _jitted():
    if not _jitted_cache:
        fmt = Format(
            Layout(major_to_minor=(2, 1, 0), tiling=((8, 128),)),
            jax.sharding.SingleDeviceSharding(jax.devices()[0]),
        )
        _jitted_cache.append(jax.jit(_kernel_impl, out_shardings=fmt))
    return _jitted_cache[0]


def kernel(seq_inputs, e1_pos_inputs, e2_pos_inputs, we, wpe):
    return _jitted()(seq_inputs, e1_pos_inputs, e2_pos_inputs, we, wpe)


# speedup vs baseline: 1.0016x; 1.0016x over previous
import jax
import jax.numpy as jnp
from jax import lax
from jax.experimental import pallas as pl
from jax.experimental.pallas import tpu as pltpu
from jax.experimental.pallas import tpu_sc as plsc
from jax.experimental.layout import Format, Layout

B = 4096
L = 200
NE = 2
J = L + 2 * NE
H0 = 104
H1 = L - H0
DW = 64
DP = 16
DOUT = DW + 2 * DP
RB = 16

_info = plsc.get_sparse_core_info()
NC = _info.num_cores
NS = _info.num_subcores
NW = NC * NS
ROWS_PER_W = B // NW


def _body(seq_hbm, e1_hbm, e2_hbm, we_hbm, wpe_hbm, out_hbm,
          seq_rows, e1_rows, e2_rows, we0,
          wb0, pb10, pb20, wb1, pb11, pb21,
          sg0, sg1, sw0, sw1):
    sets = ((wb0, pb10, pb20, sg0, sw0),
            (wb1, pb11, pb21, sg1, sw1))
    wid = lax.axis_index("s") * NC + lax.axis_index("c")
    b0 = wid * ROWS_PER_W

    pltpu.sync_copy(we_hbm.at[pl.ds(0, 1)], we0)
    for k in range(DW // 16):
        v = we0[0, pl.ds(k * 16, 16)]
        for wbx in (wb0, wb1):
            for rr in (0, 1, J - 2, J - 1):
                wbx[rr, pl.ds(k * 16, 16)] = v

    def gathers(s, r):
        wbx, p1x, p2x, sg = sets[s][0], sets[s][1], sets[s][2], sets[s][3]
        g = lax.rem(r, RB)
        cps = []
        for (off, n) in ((0, H0), (H0, H1)):
            cps.append(pltpu.make_async_copy(
                we_hbm.at[seq_rows.at[g, pl.ds(off, n)]],
                wbx.at[pl.ds(NE + off, n)], sg))
            cps.append(pltpu.make_async_copy(
                wpe_hbm.at[e1_rows.at[g, pl.ds(off, n)]],
                p1x.at[pl.ds(NE + off, n)], sg))
            cps.append(pltpu.make_async_copy(
                wpe_hbm.at[e2_rows.at[g, pl.ds(off, n)]],
                p2x.at[pl.ds(NE + off, n)], sg))
        return cps

    def fixup(s):
        for px in (sets[s][1], sets[s][2]):
            v = px[NE]
            px[0] = v
            px[1] = v
            v = px[J - 3]
            px[J - 2] = v
            px[J - 1] = v

    def writes(s, b):
        wbx, p1x, p2x, sw = sets[s][0], sets[s][1], sets[s][2], sets[s][4]
        return [
            pltpu.make_async_copy(wbx, out_hbm.at[b, :, pl.ds(0, DW)], sw),
            pltpu.make_async_copy(p1x, out_hbm.at[b, :, pl.ds(DW, DP)], sw),
            pltpu.make_async_copy(p2x, out_hbm.at[b, :, pl.ds(DW + DP, DP)],
                                  sw),
        ]

    def stage_rows(r):
        @pl.when(lax.rem(r, RB) == 0)
        def _():
            base = pl.multiple_of(b0 + r, RB)
            pltpu.sync_copy(seq_hbm.at[pl.ds(base, RB)], seq_rows)
            pltpu.sync_copy(e1_hbm.at[pl.ds(base, RB)], e1_rows)
            pltpu.sync_copy(e2_hbm.at[pl.ds(base, RB)], e2_rows)

    def half(r, s, so, do_waitw):
        for cp in gathers(s, r):
            cp.wait()
        fixup(s)
        for cp in writes(s, b0 + r):
            cp.start()
        stage_rows(r + 1)
        if do_waitw:
            for cp in writes(so, b0 + r - 1):
                cp.wait()
        for cp in gathers(so, r + 1):
            cp.start()

    stage_rows(0)
    for cp in gathers(0, 0):
        cp.start()
    half(0, 0, 1, do_waitw=False)

    def pair(i, carry):
        r = 2 * i + 1
        half(r, 1, 0, do_waitw=True)
        half(r + 1, 0, 1, do_waitw=True)
        return carry

    lax.fori_loop(0, (ROWS_PER_W - 2) // 2, pair, 0)

    rlast = ROWS_PER_W - 1
    for cp in gathers(1, rlast):
        cp.wait()
    fixup(1)
    for cp in writes(1, b0 + rlast):
        cp.start()
    for cp in writes(0, b0 + rlast - 1):
        cp.wait()
    for cp in writes(1, b0 + rlast):
        cp.wait()


def _kernel_impl(seq_inputs, e1_pos_inputs, e2_pos_inputs, we, wpe):
    run = pl.kernel(
        _body,
        mesh=plsc.VectorSubcoreMesh(core_axis_name="c", subcore_axis_name="s"),
        compiler_params=pltpu.CompilerParams(use_tc_tiling_on_sc=False,
                                             needs_layout_passes=False),
        out_type=jax.ShapeDtypeStruct((B, J, DOUT), jnp.float32),
        scratch_types=[
            pltpu.VMEM((RB, L), jnp.int32),
            pltpu.VMEM((RB, L), jnp.int32),
            pltpu.VMEM((RB, L), jnp.int32),
            pltpu.VMEM((1, DW), jnp.float32),
        ] + 2 * [
            pltpu.VMEM((J, DW), jnp.float32),
            pltpu.VMEM((J, DP), jnp.float32),
            pltpu.VMEM((J, DP), jnp.float32),
        ] + 4 * [pltpu.SemaphoreType.DMA],
    )
    return run(seq_inputs.astype(jnp.int32),
               e1_pos_inputs.astype(jnp.int32),
               e2_pos_inputs.astype(jnp.int32),
               we, wpe)


_jitted_cache = []


def _jitted():
    if not _jitted_cache:
        fmt = Format(
            Layout(major_to_minor=(2, 1, 0), tiling=((8,),)),
            jax.sharding.SingleDeviceSharding(jax.devices()[0]),
        )
        _jitted_cache.append(jax.jit(_kernel_impl, out_shardings=fmt))
    return _jitted_cache[0]


def kernel(seq_inputs, e1_pos_inputs, e2_pos_inputs, we, wpe):
    return _jitted()(seq_inputs, e1_pos_inputs, e2_pos_inputs, we, wpe)
